# Initial kernel scaffold; baseline (speedup 1.0000x reference)
import jax
import jax.numpy as jnp
from jax import lax
from jax.experimental import pallas as pl
from jax.experimental.pallas import tpu as pltpu
from jax.experimental.pallas import tpu_sc as plsc

N = 2048
F_IN = 128
H = 16
E = 65536
KF = N * (H - 2)

NC = 2
NS = 16
NW = NC * NS
CHUNK = 128
NCH = E // NW // CHUNK

_sc_mesh = plsc.VectorSubcoreMesh(core_axis_name="c", subcore_axis_name="s")


def _deg_body(dst_hbm, ones_hbm, zeros_hbm, out_hbm, dst_v, ones_v, deg_sh):
    c = lax.axis_index("c")
    s = lax.axis_index("s")
    wid = s * NC + c
    pltpu.sync_copy(dst_hbm.at[pl.ds(wid * NCH, NCH)], dst_v)
    pltpu.sync_copy(ones_hbm, ones_v)

    @pl.when(s == 0)
    def _zero():
        pltpu.sync_copy(zeros_hbm, deg_sh)

    plsc.subcore_barrier()
    for j in range(NCH):
        pltpu.sync_copy(ones_v, deg_sh.at[dst_v.at[j]], add=True)
    plsc.subcore_barrier()

    @pl.when(s == 0)
    def _out():
        pltpu.sync_copy(deg_sh, out_hbm.at[c])


_deg_kernel = pl.kernel(
    _deg_body,
    out_type=jax.ShapeDtypeStruct((NC, N, H), jnp.float32),
    mesh=_sc_mesh,
    scratch_types=[
        pltpu.VMEM((NCH, CHUNK), jnp.int32),
        pltpu.VMEM((CHUNK, H), jnp.float32),
        pltpu.VMEM_SHARED((N, H), jnp.float32),
    ],
)


def _agg_body(hd_hbm, src_hbm, dst_hbm, zeros_hbm, out_hbm,
              src_v, dst_v, rows_v, s_sh, sem):
    c = lax.axis_index("c")
    s = lax.axis_index("s")
    wid = s * NC + c
    pltpu.sync_copy(src_hbm.at[pl.ds(wid * NCH, NCH)], src_v)
    pltpu.sync_copy(dst_hbm.at[pl.ds(wid * NCH, NCH)], dst_v)

    @pl.when(s == 0)
    def _zero():
        pltpu.sync_copy(zeros_hbm, s_sh)

    copies = [pltpu.async_copy(hd_hbm.at[src_v.at[j]], rows_v.at[j], sem)
              for j in range(NCH)]
    plsc.subcore_barrier()
    for cp in copies:
        cp.wait()
    for j in range(NCH):
        pltpu.sync_copy(rows_v.at[j], s_sh.at[dst_v.at[j]], add=True)
    plsc.subcore_barrier()

    @pl.when(s == 0)
    def _out():
        pltpu.sync_copy(s_sh, out_hbm.at[c])


_agg_kernel = pl.kernel(
    _agg_body,
    out_type=jax.ShapeDtypeStruct((NC, N, H), jnp.float32),
    mesh=_sc_mesh,
    scratch_types=[
        pltpu.VMEM((NCH, CHUNK), jnp.int32),
        pltpu.VMEM((NCH, CHUNK), jnp.int32),
        pltpu.VMEM((NCH, CHUNK, H), jnp.float32),
        pltpu.VMEM_SHARED((N, H), jnp.float32),
        pltpu.SemaphoreType.DMA,
    ],
)


def _prelude_body(x_ref, w_ref, degp_ref, hd_ref, dinv_ref):
    h = jnp.dot(x_ref[...], w_ref[...], preferred_element_type=jnp.float32)
    deg = degp_ref[0] + degp_ref[1] + 1.0
    dinv = lax.rsqrt(deg)
    hd_ref[...] = dinv * h
    dinv_ref[...] = dinv


_prelude = pl.pallas_call(
    _prelude_body,
    out_shape=(jax.ShapeDtypeStruct((N, H), jnp.float32),
               jax.ShapeDtypeStruct((N, H), jnp.float32)),
)


def _finalize_body(sp_ref, hd_ref, dinv_ref, bg_ref, wt_ref, bt_ref, a_ref):
    s_tot = sp_ref[0] + sp_ref[1]
    g = jnp.maximum(dinv_ref[...] * (s_tot + hd_ref[...]) + bg_ref[...], 0.0)
    conv = (wt_ref[0] * g[:, 0:H - 2] + wt_ref[1] * g[:, 1:H - 1]
            + wt_ref[2] * g[:, 2:H]) + bt_ref[0]
    a_ref[...] = jnp.maximum(conv, 0.0)


_finalize = pl.pallas_call(
    _finalize_body,
    in_specs=[
        pl.BlockSpec((NC, N, H), lambda: (0, 0, 0)),
        pl.BlockSpec((N, H), lambda: (0, 0)),
        pl.BlockSpec((N, H), lambda: (0, 0)),
        pl.BlockSpec((1, H), lambda: (0, 0)),
        pl.BlockSpec(memory_space=pltpu.SMEM),
        pl.BlockSpec(memory_space=pltpu.SMEM),
    ],
    out_shape=jax.ShapeDtypeStruct((N, H - 2), jnp.float32),
)


GN = 16
NB = N // GN


def _fc_body(a_ref, w_ref, b_ref, o_ref):
    prod = w_ref[...] * a_ref[...]
    s1 = jnp.sum(prod.reshape(NB, KF // 128, 128), axis=1)
    o_ref[...] = jnp.sum(s1, axis=1, keepdims=True) + b_ref[...]


_fc = pl.pallas_call(
    _fc_body,
    grid=(GN,),
    in_specs=[
        pl.BlockSpec((1, KF), lambda i: (0, 0)),
        pl.BlockSpec((NB, KF), lambda i: (i, 0)),
        pl.BlockSpec((NB, 1), lambda i: (i, 0)),
    ],
    out_specs=pl.BlockSpec((NB, 1), lambda i: (i, 0)),
    out_shape=jax.ShapeDtypeStruct((N, 1), jnp.float32),
)


def kernel(x, edge_index, W_gcn, b_gcn, w_tcn, b_tcn, W_fc, b_fc):
    src2 = edge_index[0].reshape(E // CHUNK, CHUNK)
    dst2 = edge_index[1].reshape(E // CHUNK, CHUNK)
    ones = jnp.ones((CHUNK, H), jnp.float32)
    zeros = jnp.zeros((N, H), jnp.float32)

    degp = _deg_kernel(dst2, ones, zeros)
    hd, dinv = _prelude(x, W_gcn, degp)
    sp = _agg_kernel(hd, src2, dst2, zeros)
    a = _finalize(sp, hd, dinv, b_gcn.reshape(1, H), w_tcn, b_tcn)
    flat = a.reshape(1, KF)
    out = _fc(flat, W_fc, b_fc.reshape(N, 1))
    return out.reshape(1, N)


# speedup vs baseline: 11.6458x; 11.6458x over previous
import jax
import jax.numpy as jnp
from jax import lax
from jax.experimental import pallas as pl
from jax.experimental.pallas import tpu as pltpu
from jax.experimental.pallas import tpu_sc as plsc

N = 2048
F_IN = 128
H = 16
E = 65536
KF = N * (H - 2)

NC = 2
NS = 16
NW = NC * NS
CHUNK = 128
NCH = E // NW // CHUNK

_sc_kernels_cache = []


def _sc_kernels():
    if _sc_kernels_cache:
        return _sc_kernels_cache[0]
    mesh = plsc.VectorSubcoreMesh(core_axis_name="c", subcore_axis_name="s",
                                  num_cores=NC, num_subcores=NS)
    params = pltpu.CompilerParams(use_tc_tiling_on_sc=False)
    deg_kernel = pl.kernel(
        _deg_body,
        out_type=jax.ShapeDtypeStruct((NC, N, H), jnp.float32),
        mesh=mesh,
        compiler_params=params,
        scratch_types=[
            pltpu.VMEM((NCH, CHUNK), jnp.int32),
            pltpu.VMEM((CHUNK, H), jnp.float32),
            pltpu.VMEM_SHARED((N, H), jnp.float32),
        ],
    )
    agg_kernel = pl.kernel(
        _agg_body,
        out_type=jax.ShapeDtypeStruct((NC, N, H), jnp.float32),
        mesh=mesh,
        compiler_params=params,
        scratch_types=[
            pltpu.VMEM((NCH, CHUNK), jnp.int32),
            pltpu.VMEM((NCH, CHUNK), jnp.int32),
            pltpu.VMEM((NCH, CHUNK, H), jnp.float32),
            pltpu.VMEM_SHARED((N, H), jnp.float32),
            pltpu.SemaphoreType.DMA,
        ],
    )
    _sc_kernels_cache.append((deg_kernel, agg_kernel))
    return _sc_kernels_cache[0]


def _deg_body(dst_hbm, ones_hbm, zeros_hbm, out_hbm, dst_v, ones_v, deg_sh):
    c = lax.axis_index("c")
    s = lax.axis_index("s")
    wid = s * NC + c
    pltpu.sync_copy(dst_hbm.at[pl.ds(wid * NCH, NCH)], dst_v)
    pltpu.sync_copy(ones_hbm, ones_v)

    @pl.when(s == 0)
    def _zero():
        pltpu.sync_copy(zeros_hbm, deg_sh)

    plsc.subcore_barrier()
    for j in range(NCH):
        pltpu.sync_copy(ones_v, deg_sh.at[dst_v.at[j]], add=True)
    plsc.subcore_barrier()

    @pl.when(s == 0)
    def _out():
        pltpu.sync_copy(deg_sh, out_hbm.at[c])


def _agg_body(hd_hbm, src_hbm, dst_hbm, zeros_hbm, out_hbm,
              src_v, dst_v, rows_v, s_sh, sem):
    c = lax.axis_index("c")
    s = lax.axis_index("s")
    wid = s * NC + c
    pltpu.sync_copy(src_hbm.at[pl.ds(wid * NCH, NCH)], src_v)
    pltpu.sync_copy(dst_hbm.at[pl.ds(wid * NCH, NCH)], dst_v)

    @pl.when(s == 0)
    def _zero():
        pltpu.sync_copy(zeros_hbm, s_sh)

    copies = [pltpu.async_copy(hd_hbm.at[src_v.at[j]], rows_v.at[j], sem)
              for j in range(NCH)]
    plsc.subcore_barrier()
    for cp in copies:
        cp.wait()
    for j in range(NCH):
        pltpu.sync_copy(rows_v.at[j], s_sh.at[dst_v.at[j]], add=True)
    plsc.subcore_barrier()

    @pl.when(s == 0)
    def _out():
        pltpu.sync_copy(s_sh, out_hbm.at[c])


def _prelude_body(x_ref, w_ref, degp_ref, hd_ref, dinv_ref):
    h = jnp.dot(x_ref[...], w_ref[...], preferred_element_type=jnp.float32)
    deg = degp_ref[0] + degp_ref[1] + 1.0
    dinv = lax.rsqrt(deg)
    hd_ref[...] = dinv * h
    dinv_ref[...] = dinv


_prelude = pl.pallas_call(
    _prelude_body,
    out_shape=(jax.ShapeDtypeStruct((N, H), jnp.float32),
               jax.ShapeDtypeStruct((N, H), jnp.float32)),
)


def _finalize_body(sp_ref, hd_ref, dinv_ref, bg_ref, wt_ref, bt_ref, a_ref):
    s_tot = sp_ref[0] + sp_ref[1]
    g = jnp.maximum(dinv_ref[...] * (s_tot + hd_ref[...]) + bg_ref[...], 0.0)
    conv = (wt_ref[0] * g[:, 0:H - 2] + wt_ref[1] * g[:, 1:H - 1]
            + wt_ref[2] * g[:, 2:H]) + bt_ref[0]
    a_ref[...] = jnp.maximum(conv, 0.0)


_finalize = pl.pallas_call(
    _finalize_body,
    in_specs=[
        pl.BlockSpec((NC, N, H), lambda: (0, 0, 0)),
        pl.BlockSpec((N, H), lambda: (0, 0)),
        pl.BlockSpec((N, H), lambda: (0, 0)),
        pl.BlockSpec((1, H), lambda: (0, 0)),
        pl.BlockSpec(memory_space=pltpu.SMEM),
        pl.BlockSpec(memory_space=pltpu.SMEM),
    ],
    out_shape=jax.ShapeDtypeStruct((N, H - 2), jnp.float32),
)


GN = 16
NB = N // GN


def _fc_body(a_ref, w_ref, b_ref, o_ref):
    prod = w_ref[...] * a_ref[...]
    s1 = jnp.sum(prod.reshape(NB, KF // 128, 128), axis=1)
    o_ref[...] = jnp.sum(s1, axis=1, keepdims=True) + b_ref[...]


_fc = pl.pallas_call(
    _fc_body,
    grid=(GN,),
    in_specs=[
        pl.BlockSpec((1, KF), lambda i: (0, 0)),
        pl.BlockSpec((NB, KF), lambda i: (i, 0)),
        pl.BlockSpec((NB, 1), lambda i: (i, 0)),
    ],
    out_specs=pl.BlockSpec((NB, 1), lambda i: (i, 0)),
    out_shape=jax.ShapeDtypeStruct((N, 1), jnp.float32),
)


def kernel(x, edge_index, W_gcn, b_gcn, w_tcn, b_tcn, W_fc, b_fc):
    src2 = edge_index[0].reshape(E // CHUNK, CHUNK)
    dst2 = edge_index[1].reshape(E // CHUNK, CHUNK)
    ones = jnp.ones((CHUNK, H), jnp.float32)
    zeros = jnp.zeros((N, H), jnp.float32)

    deg_kernel, agg_kernel = _sc_kernels()
    degp = deg_kernel(dst2, ones, zeros)
    hd, dinv = _prelude(x, W_gcn, degp)
    sp = agg_kernel(hd, src2, dst2, zeros)
    a = _finalize(sp, hd, dinv, b_gcn.reshape(1, H), w_tcn, b_tcn)
    flat = a.reshape(1, KF)
    out = _fc(flat, W_fc, b_fc.reshape(N, 1))
    return out.reshape(1, N)
